# baseline (speedup 1.0000x reference)
import jax
import jax.numpy as jnp
from jax import lax
from jax.experimental import pallas as pl
from jax.experimental.pallas import tpu as pltpu
from jax.experimental.pallas import tpu_sc as plsc

_N_ROUTED = 7
_LANES = 128
_T = 512
_T2 = 1024
_NC = 2
_NS = 16
_NW = _NC * _NS


def _gelu(v):
    return 0.5 * v * (1.0 + jax.lax.erf(v * 0.7071067811865476))


def _router_body(x_ref, wg_ref, l_ref, pos0_ref, pos1_ref, w0_ref, w1_ref,
                 offs_ref, oh0_scr, oh1_scr, w_scr, carry_scr, snap_scr):
    i = pl.program_id(0)
    n_tiles = pl.num_programs(0) // 2

    @pl.when(i == 0)
    def _init():
        carry_scr[...] = jnp.zeros_like(carry_scr)

    @pl.when(i < n_tiles)
    def _pass1():
        xt = x_ref[...]
        logits = jax.lax.dot_general(
            xt, wg_ref[...], (((1,), (0,)), ((), ())),
            preferred_element_type=jnp.float32)
        lane = jax.lax.broadcasted_iota(jnp.int32, logits.shape, 1)
        neg = jnp.float32(-1e30)
        logits = jnp.where(lane < _N_ROUTED, logits, neg)
        m0 = jnp.max(logits, axis=1, keepdims=True)
        idx0 = jnp.min(jnp.where(logits == m0, lane, _LANES), axis=1,
                       keepdims=True)
        logits1 = jnp.where(lane == idx0, neg, logits)
        m1 = jnp.max(logits1, axis=1, keepdims=True)
        idx1 = jnp.min(jnp.where(logits1 == m1, lane, _LANES), axis=1,
                       keepdims=True)
        e1 = jnp.exp(m1 - m0)
        w0 = 1.0 / (1.0 + e1)
        w1 = e1 * w0
        oh0 = (lane == idx0).astype(jnp.float32)
        oh1 = (lane == idx1).astype(jnp.float32)
        sl = pl.ds(i * _T, _T)
        oh0_scr[sl, :] = oh0
        oh1_scr[sl, :] = oh1
        w_scr[sl, :] = jnp.concatenate([w0, w1], axis=1)
        snap_scr[pl.ds(i, 1), :] = carry_scr[...]
        carry_scr[...] = carry_scr[...] + jnp.sum(oh0 + oh1, axis=0,
                                                  keepdims=True)

    @pl.when(i >= n_tiles)
    def _pass2():
        k = i - n_tiles
        sl = pl.ds(k * _T, _T)
        oh0 = oh0_scr[sl, :]
        oh1 = oh1_scr[sl, :]
        ohsum = (oh0 + oh1).astype(jnp.bfloat16)
        lmm = jax.lax.dot_general(
            l_ref[...], ohsum, (((1,), (0,)), ((), ())),
            preferred_element_type=jnp.float32)
        totals = carry_scr[...]
        base = jnp.zeros_like(totals)
        for s in range(1, _N_ROUTED + 1):
            base = base + pltpu.roll(totals, s, axis=1)
        prior = lmm + snap_scr[pl.ds(k, 1), :] + base
        pos0 = jnp.sum(prior * oh0, axis=1, keepdims=True)
        pos1 = jnp.sum((prior + oh0) * oh1, axis=1, keepdims=True)
        pos0_ref[...] = pos0.astype(jnp.int32)
        pos1_ref[...] = pos1.astype(jnp.int32)
        w0_ref[...] = w_scr[sl, 0:1]
        w1_ref[...] = w_scr[sl, 1:2]
        offs_ref[...] = base.astype(jnp.int32)


def _k1_router(xb, wg_pad, ltri):
    n, d = xb.shape
    n_tiles = n // _T
    return pl.pallas_call(
        _router_body,
        grid=(2 * n_tiles,),
        in_specs=[
            pl.BlockSpec((_T, d), lambda i: (jnp.where(i < 8, i, 0), 0)),
            pl.BlockSpec((d, _LANES), lambda i: (0, 0)),
            pl.BlockSpec((_T, _T), lambda i: (0, 0)),
        ],
        out_specs=[
            pl.BlockSpec((_T, 1), lambda i: (jnp.where(i < 8, 0, i - 8), 0)),
            pl.BlockSpec((_T, 1), lambda i: (jnp.where(i < 8, 0, i - 8), 0)),
            pl.BlockSpec((_T, 1), lambda i: (jnp.where(i < 8, 0, i - 8), 0)),
            pl.BlockSpec((_T, 1), lambda i: (jnp.where(i < 8, 0, i - 8), 0)),
            pl.BlockSpec((1, _LANES), lambda i: (0, 0)),
        ],
        out_shape=[
            jax.ShapeDtypeStruct((n, 1), jnp.int32),
            jax.ShapeDtypeStruct((n, 1), jnp.int32),
            jax.ShapeDtypeStruct((n, 1), jnp.float32),
            jax.ShapeDtypeStruct((n, 1), jnp.float32),
            jax.ShapeDtypeStruct((1, _LANES), jnp.int32),
        ],
        scratch_shapes=[
            pltpu.VMEM((n, _LANES), jnp.float32),
            pltpu.VMEM((n, _LANES), jnp.float32),
            pltpu.VMEM((n, 2), jnp.float32),
            pltpu.VMEM((1, _LANES), jnp.float32),
            pltpu.VMEM((n // _T, _LANES), jnp.float32),
        ],
    )(xb, wg_pad, ltri)


def _shared_body(x_ref, wfc_ref, bfc_ref, wproj_ref, bproj_ref, o_ref):
    xt = x_ref[...]
    h = jax.lax.dot_general(
        xt, wfc_ref[...], (((1,), (0,)), ((), ())),
        preferred_element_type=jnp.float32)
    h = _gelu(h + bfc_ref[...])
    y = jax.lax.dot_general(
        h.astype(jnp.bfloat16), wproj_ref[...], (((1,), (0,)), ((), ())),
        preferred_element_type=jnp.float32)
    o_ref[...] = y + bproj_ref[...]


def _k0_shared(xb, wfc, bfc, wproj, bproj):
    n, d = xb.shape
    hid = wfc.shape[1]
    return pl.pallas_call(
        _shared_body,
        grid=(n // _T,),
        in_specs=[
            pl.BlockSpec((_T, d), lambda i: (i, 0)),
            pl.BlockSpec((d, hid), lambda i: (0, 0)),
            pl.BlockSpec((1, hid), lambda i: (0, 0)),
            pl.BlockSpec((hid, d), lambda i: (0, 0)),
            pl.BlockSpec((1, d), lambda i: (0, 0)),
        ],
        out_specs=pl.BlockSpec((_T, d), lambda i: (i, 0)),
        out_shape=jax.ShapeDtypeStruct((n, d), jnp.float32),
    )(xb, wfc, bfc[None], wproj, bproj[None])


def _grouped_body(offs_ref, xs_ref, wfc_ref, bfc_ref, wproj_ref, bproj_ref,
                  ys_ref):
    i = pl.program_id(0)
    row0 = i * _T2
    rows = row0 + jax.lax.broadcasted_iota(jnp.int32, (_T2, 1), 0)
    xb = xs_ref[...].astype(jnp.bfloat16)
    ys_ref[...] = jnp.zeros_like(ys_ref)
    for e in range(_N_ROUTED):
        start = offs_ref[e]
        end = offs_ref[e + 1]

        @pl.when((start < row0 + _T2) & (end > row0))
        def _run(e=e, start=start, end=end):
            h = jax.lax.dot_general(
                xb, wfc_ref[e], (((1,), (0,)), ((), ())),
                preferred_element_type=jnp.float32)
            h = _gelu(h + bfc_ref[e, :])
            y = jax.lax.dot_general(
                h.astype(jnp.bfloat16), wproj_ref[e], (((1,), (0,)), ((), ())),
                preferred_element_type=jnp.float32)
            y = y + bproj_ref[e, :]
            mask = ((rows >= start) & (rows < end)).astype(jnp.float32)
            ys_ref[...] = ys_ref[...] + mask * y


def _k3_grouped(offs, xs, wfc, bfc, wproj, bproj):
    m, d = xs.shape
    hid = wfc.shape[2]
    return pl.pallas_call(
        _grouped_body,
        grid_spec=pltpu.PrefetchScalarGridSpec(
            num_scalar_prefetch=1,
            grid=(m // _T2,),
            in_specs=[
                pl.BlockSpec((_T2, d), lambda i, offs: (i, 0)),
                pl.BlockSpec((_N_ROUTED, d, hid), lambda i, offs: (0, 0, 0)),
                pl.BlockSpec((_N_ROUTED, hid), lambda i, offs: (0, 0)),
                pl.BlockSpec((_N_ROUTED, hid, d), lambda i, offs: (0, 0, 0)),
                pl.BlockSpec((_N_ROUTED, d), lambda i, offs: (0, 0)),
            ],
            out_specs=pl.BlockSpec((_T2, d), lambda i, offs: (i, 0)),
        ),
        out_shape=jax.ShapeDtypeStruct((m, d), jnp.float32),
    )(offs, xs, wfc, bfc, wproj, bproj)


_DSUB = 64


def _dispatch_body(x_hbm, pos0_hbm, pos1_hbm, xs_hbm, idx0, idx1, xblk, sem):
    wid = lax.axis_index("s") * _NC + lax.axis_index("c")
    n_tok = x_hbm.shape[0]
    tok_w = n_tok // _NW
    for k in range(tok_w // _DSUB):
        tok0 = wid * tok_w + k * _DSUB
        pltpu.sync_copy(x_hbm.at[pl.ds(tok0, _DSUB)], xblk)
        pltpu.sync_copy(pos0_hbm.at[pl.ds(tok0, _DSUB)], idx0)
        pltpu.sync_copy(pos1_hbm.at[pl.ds(tok0, _DSUB)], idx1)
        c0 = pltpu.async_copy(xblk, xs_hbm.at[idx0], sem)
        c1 = pltpu.async_copy(xblk, xs_hbm.at[idx1], sem)
        c0.wait()
        c1.wait()


def _sc1_dispatch(xf, pos0, pos1):
    n, d = xf.shape
    mesh = plsc.VectorSubcoreMesh(core_axis_name="c", subcore_axis_name="s")
    return pl.kernel(
        _dispatch_body,
        out_type=jax.ShapeDtypeStruct((2 * n, d), jnp.float32),
        mesh=mesh,
        scratch_types=[
            pltpu.VMEM((_DSUB,), jnp.int32),
            pltpu.VMEM((_DSUB,), jnp.int32),
            pltpu.VMEM((_DSUB, d), jnp.float32),
            pltpu.SemaphoreType.DMA,
        ],
    )(xf, pos0, pos1)


_CSUB = 16


def _combine_body(ys_hbm, pos0_hbm, pos1_hbm, w0_hbm, w1_hbm, sh_hbm, out_hbm,
                  p0v, p1v, w0v, w1v, rows0, rows1, shblk, oblk, sem):
    wid = lax.axis_index("s") * _NC + lax.axis_index("c")
    n_tok = sh_hbm.shape[0]
    tok_w = n_tok // _NW
    for k in range(tok_w // _CSUB):
        tok0 = wid * tok_w + k * _CSUB
        pltpu.sync_copy(sh_hbm.at[pl.ds(tok0, _CSUB)], shblk)
        pltpu.sync_copy(pos0_hbm.at[pl.ds(tok0, _CSUB)], p0v)
        pltpu.sync_copy(pos1_hbm.at[pl.ds(tok0, _CSUB)], p1v)
        pltpu.sync_copy(w0_hbm.at[pl.ds(tok0, _CSUB)], w0v)
        pltpu.sync_copy(w1_hbm.at[pl.ds(tok0, _CSUB)], w1v)
        g0 = pltpu.async_copy(ys_hbm.at[p0v], rows0, sem)
        g1 = pltpu.async_copy(ys_hbm.at[p1v], rows1, sem)
        g0.wait()
        g1.wait()
        wa = w0v[...]
        wb = w1v[...]
        for t in range(_CSUB):
            w0 = jnp.broadcast_to(jax.lax.slice(wa, (t,), (t + 1,)), (16,))
            w1 = jnp.broadcast_to(jax.lax.slice(wb, (t,), (t + 1,)), (16,))

            def body(g, carry, t=t, w0=w0, w1=w1):
                slg = pl.ds(g * 16, 16)
                oblk[t, slg] = (shblk[t, slg] + w0 * rows0[t, slg]
                                + w1 * rows1[t, slg])
                return carry

            lax.fori_loop(0, oblk.shape[1] // 16, body, 0, unroll=4)
        pltpu.sync_copy(oblk, out_hbm.at[pl.ds(tok0, _CSUB)])


def _sc2_combine(ys, pos0, pos1, w0, w1, shared):
    n, d = shared.shape
    mesh = plsc.VectorSubcoreMesh(core_axis_name="c", subcore_axis_name="s")
    return pl.kernel(
        _combine_body,
        out_type=jax.ShapeDtypeStruct((n, d), jnp.float32),
        mesh=mesh,
        scratch_types=[
            pltpu.VMEM((_CSUB,), jnp.int32),
            pltpu.VMEM((_CSUB,), jnp.int32),
            pltpu.VMEM((_CSUB,), jnp.float32),
            pltpu.VMEM((_CSUB,), jnp.float32),
            pltpu.VMEM((_CSUB, d), jnp.float32),
            pltpu.VMEM((_CSUB, d), jnp.float32),
            pltpu.VMEM((_CSUB, d), jnp.float32),
            pltpu.VMEM((_CSUB, d), jnp.float32),
            pltpu.SemaphoreType.DMA,
        ],
    )(ys, pos0, pos1, w0, w1, shared)


@jax.jit
def kernel(x, Ws_fc, bs_fc, Ws_proj, bs_proj, Wr_fc, br_fc, Wr_proj, br_proj, Wg):
    B, S, D = x.shape
    N = B * S
    M = N * 2
    xf = x.reshape(N, D)
    xb = xf.astype(jnp.bfloat16)

    wg = jnp.pad(Wg, ((0, 0), (0, _LANES - Wg.shape[1]))).astype(jnp.bfloat16)
    ltri = jnp.tril(jnp.ones((_T, _T), jnp.bfloat16), k=-1)

    pos0, pos1, w0, w1, offs_row = _k1_router(xb, wg, ltri)
    shared = _k0_shared(xb, Ws_fc.astype(jnp.bfloat16), bs_fc,
                        Ws_proj.astype(jnp.bfloat16), bs_proj)

    xs = _sc1_dispatch(xf, pos0.reshape(N), pos1.reshape(N))

    offs = jnp.concatenate(
        [offs_row[0, :_N_ROUTED], jnp.array([M], jnp.int32)])
    ys = _k3_grouped(offs, xs,
                     Wr_fc.astype(jnp.bfloat16), br_fc,
                     Wr_proj.astype(jnp.bfloat16), br_proj)

    out = _sc2_combine(ys, pos0.reshape(N), pos1.reshape(N),
                       w0.reshape(N), w1.reshape(N), shared)
    return out.reshape(B, S, D)


# speedup vs baseline: 1.3901x; 1.3901x over previous
import jax
import jax.numpy as jnp
from jax.experimental import pallas as pl
from jax.experimental.pallas import tpu as pltpu

_N_EXPERTS = 8
_N_ROUTED = 7
_LANES = 128


def _gelu(v):
    return 0.5 * v * (1.0 + jax.lax.erf(v * 0.7071067811865476))


def _moe_body(x_ref, wg_ref, wfc_ref, bfc_ref, wprojs_ref, bproj_ref, o_ref):
    xb = x_ref[...]

    logits = jax.lax.dot_general(
        xb, wg_ref[...], (((1,), (0,)), ((), ())),
        preferred_element_type=jnp.float32)
    lane = jax.lax.broadcasted_iota(jnp.int32, logits.shape, 1)
    neg = jnp.float32(-1e30)
    logits = jnp.where(lane < _N_ROUTED, logits, neg)

    m0 = jnp.max(logits, axis=1, keepdims=True)
    idx0 = jnp.min(jnp.where(logits == m0, lane, _LANES), axis=1,
                   keepdims=True)
    logits1 = jnp.where(lane == idx0, neg, logits)
    m1 = jnp.max(logits1, axis=1, keepdims=True)
    idx1 = jnp.min(jnp.where(logits1 == m1, lane, _LANES), axis=1,
                   keepdims=True)
    e1 = jnp.exp(m1 - m0)
    w0 = 1.0 / (1.0 + e1)
    w1 = e1 * w0

    parts = []
    gates = []
    for j in range(_N_EXPERTS):
        h = jax.lax.dot_general(
            xb, wfc_ref[j], (((1,), (0,)), ((), ())),
            preferred_element_type=jnp.float32)
        h = _gelu(h + bfc_ref[j, :])
        if j == 0:
            parts.append(h.astype(jnp.bfloat16))
        else:
            gate = w0 * (idx0 == j - 1) + w1 * (idx1 == j - 1)
            parts.append((gate * h).astype(jnp.bfloat16))
            gates.append(gate)
    hcat = jnp.concatenate(parts, axis=1)

    y = jax.lax.dot_general(
        hcat, wprojs_ref[...], (((1,), (0,)), ((), ())),
        preferred_element_type=jnp.float32)

    yb = bproj_ref[0, :]
    for j in range(_N_ROUTED):
        yb = yb + gates[j] * bproj_ref[j + 1, :]
    o_ref[...] = y + yb


@jax.jit
def kernel(x, Ws_fc, bs_fc, Ws_proj, bs_proj, Wr_fc, br_fc, Wr_proj, br_proj, Wg):
    B, S, D = x.shape
    N = B * S
    HID = Ws_fc.shape[1]
    xb = x.reshape(N, D).astype(jnp.bfloat16)

    wfc = jnp.concatenate([Ws_fc[None], Wr_fc], axis=0).astype(jnp.bfloat16)
    wprojs = jnp.concatenate([Ws_proj, Wr_proj.reshape(_N_ROUTED * HID, D)],
                             axis=0).astype(jnp.bfloat16)
    bfc = jnp.concatenate([bs_fc[None], br_fc], axis=0)
    bproj = jnp.concatenate([bs_proj[None], br_proj], axis=0)
    wg = jnp.pad(Wg, ((0, 0), (0, _LANES - Wg.shape[1]))).astype(jnp.bfloat16)

    T = 512
    grid = (N // T,)
    out = pl.pallas_call(
        _moe_body,
        grid=grid,
        in_specs=[
            pl.BlockSpec((T, D), lambda i: (i, 0)),
            pl.BlockSpec((D, _LANES), lambda i: (0, 0)),
            pl.BlockSpec((_N_EXPERTS, D, HID), lambda i: (0, 0, 0)),
            pl.BlockSpec((_N_EXPERTS, HID), lambda i: (0, 0)),
            pl.BlockSpec((_N_EXPERTS * HID, D), lambda i: (0, 0)),
            pl.BlockSpec((_N_EXPERTS, D), lambda i: (0, 0)),
        ],
        out_specs=pl.BlockSpec((T, D), lambda i: (i, 0)),
        out_shape=jax.ShapeDtypeStruct((N, D), jnp.float32),
    )(xb, wg, wfc, bfc, wprojs, bproj)
    return out.reshape(B, S, D)
